# baseline (speedup 1.0000x reference)
import functools

import jax
import jax.numpy as jnp
from jax import lax
from jax.experimental import pallas as pl
from jax.experimental.pallas import tpu as pltpu
from jax.experimental.pallas import tpu_sc as plsc

N = 10000
E = 320000
D = 128
DW = 16
NC, NS = 2, 16
NW = NC * NS
B = 112
NB = 93
EPW = NB * B
EP = NW * EPW
RPS = 640
NP = NS * RPS

_mesh = plsc.VectorSubcoreMesh(
    core_axis_name="c", subcore_axis_name="s", num_cores=NC, num_subcores=NS)


@functools.partial(
    pl.kernel,
    out_type=jax.ShapeDtypeStruct((NC, NP), jnp.float32),
    mesh=_mesh,
    scratch_types=[
        pltpu.VMEM_SHARED((NP,), jnp.float32),
        pltpu.VMEM((4, B), jnp.int32),
        pltpu.VMEM((B,), jnp.float32),
        pltpu.SemaphoreType.DMA,
        pltpu.SemaphoreType.DMA,
        pltpu.SemaphoreType.DMA,
        pltpu.SemaphoreType.DMA,
    ],
)
def _deg_kernel(dst1d, ones_hbm, zeros_hbm, out, deg_sh, didx4, ones_v,
                isem0, isem1, ssem0, ssem1):
    c = lax.axis_index("c")
    s = lax.axis_index("s")
    w = c * NS + s
    isems = (isem0, isem1)
    ssems = (ssem0, ssem1)
    pltpu.sync_copy(ones_hbm, ones_v)
    pltpu.sync_copy(zeros_hbm.at[pl.ds(s * RPS, RPS)],
                    deg_sh.at[pl.ds(s * RPS, RPS)])
    base = w * EPW

    def issue_idx(j, sem):
        pltpu.async_copy(
            dst1d.at[pl.ds(base + j * B, B)], didx4.at[lax.rem(j, 4)], sem)

    def wait_idx(j, sem):
        pltpu.make_async_copy(
            dst1d.at[pl.ds(base, B)], didx4.at[lax.rem(j, 4)], sem).wait()

    def start_scat(j, sem):
        pltpu.async_copy(
            ones_v, deg_sh.at[didx4.at[lax.rem(j, 4)]], sem, add=True)

    def drain_scat(j, sem):
        pltpu.make_async_copy(
            ones_v, deg_sh.at[didx4.at[lax.rem(j, 4)]], sem).wait()

    issue_idx(0, isem0)
    issue_idx(1, isem1)
    plsc.subcore_barrier()
    wait_idx(0, isem0)
    start_scat(0, ssem0)
    issue_idx(2, isem0)
    wait_idx(1, isem1)
    start_scat(1, ssem1)
    issue_idx(3, isem1)

    def body(j2, carry):
        for q in (0, 1):
            j = j2 * 2 + q
            jn = jnp.minimum(j + 2, NB - 1)
            drain_scat(j - 2, ssems[q])
            wait_idx(j, isems[q])
            start_scat(j, ssems[q])
            issue_idx(jn, isems[q])
        return carry

    lax.fori_loop(1, (NB - 1) // 2, body, None)
    drain_scat(NB - 3, ssems[(NB - 3) % 2])
    wait_idx(NB - 1, isems[(NB - 1) % 2])
    start_scat(NB - 1, ssems[(NB - 1) % 2])
    drain_scat(NB - 2, ssems[(NB - 2) % 2])
    drain_scat(NB - 1, ssems[(NB - 1) % 2])
    wait_idx(NB - 1, isems[(NB - 2) % 2])
    plsc.subcore_barrier()
    pltpu.sync_copy(deg_sh.at[pl.ds(s * RPS, RPS)],
                    out.at[c, pl.ds(s * RPS, RPS)])


@functools.partial(
    pl.kernel,
    out_type=jax.ShapeDtypeStruct((NC, NP, D), jnp.float32),
    mesh=_mesh,
    scratch_types=[
        pltpu.VMEM_SHARED((NP, D), jnp.float32),
        pltpu.VMEM((4, B), jnp.int32),
        pltpu.VMEM((4, B), jnp.int32),
        pltpu.VMEM((3, B, D), jnp.float32),
        pltpu.SemaphoreType.DMA,
        pltpu.SemaphoreType.DMA,
        pltpu.SemaphoreType.DMA,
        pltpu.SemaphoreType.DMA,
        pltpu.SemaphoreType.DMA,
        pltpu.SemaphoreType.DMA,
    ],
)
def _agg_kernel(src1d, dst1d, g_hbm, zeros_hbm, out, acc_sh, sidx4, didx4,
                rows3, isem0, isem1, gsem0, gsem1, ssem0, ssem1):
    c = lax.axis_index("c")
    s = lax.axis_index("s")
    w = c * NS + s
    isems = (isem0, isem1)
    gsems = (gsem0, gsem1)
    ssems = (ssem0, ssem1)
    base = w * EPW

    def issue_idx(j, sem):
        t = lax.rem(j, 4)
        pltpu.async_copy(src1d.at[pl.ds(base + j * B, B)], sidx4.at[t], sem)
        pltpu.async_copy(dst1d.at[pl.ds(base + j * B, B)], didx4.at[t], sem)

    def wait_idx(j, sem):
        t = lax.rem(j, 4)
        pltpu.make_async_copy(src1d.at[pl.ds(base, B)], sidx4.at[t],
                              sem).wait()
        pltpu.make_async_copy(dst1d.at[pl.ds(base, B)], didx4.at[t],
                              sem).wait()

    def issue_g(j, sem):
        pltpu.async_copy(
            g_hbm.at[sidx4.at[lax.rem(j, 4)]], rows3.at[lax.rem(j, 3)], sem)

    def wait_g(j, sem):
        pltpu.make_async_copy(
            g_hbm.at[sidx4.at[lax.rem(j, 4)]], rows3.at[lax.rem(j, 3)],
            sem).wait()

    def start_scat(j, sem):
        pltpu.async_copy(
            rows3.at[lax.rem(j, 3)], acc_sh.at[didx4.at[lax.rem(j, 4)]],
            sem, add=True)

    def drain_scat(j, sem):
        pltpu.make_async_copy(
            rows3.at[lax.rem(j, 3)], acc_sh.at[didx4.at[lax.rem(j, 4)]],
            sem).wait()

    issue_idx(0, isem0)
    issue_idx(1, isem1)
    pltpu.sync_copy(zeros_hbm.at[pl.ds(s * RPS, RPS)],
                    acc_sh.at[pl.ds(s * RPS, RPS)])
    plsc.subcore_barrier()

    wait_idx(0, isem0)
    issue_g(0, gsem0)
    issue_idx(2, isem0)
    wait_idx(1, isem1)
    issue_g(1, gsem1)
    issue_idx(3, isem1)
    wait_g(0, gsem0)
    start_scat(0, ssem0)
    wait_idx(2, isem0)
    issue_g(2, gsem0)
    wait_g(1, gsem1)

    def body(j2, carry):
        for q in (0, 1):
            j = j2 * 2 + q
            drain_scat(j - 2, ssems[q])
            start_scat(j - 1, ssems[1 - q])
            wait_idx(j + 1, isems[1 - q])
            issue_g(j + 1, gsems[1 - q])
            issue_idx(jnp.minimum(j + 2, NB - 1), isems[q])
            wait_g(j, gsems[q])
        return carry

    lax.fori_loop(1, (NB - 1) // 2, body, None)
    drain_scat(NB - 3, ssems[(NB - 3) % 2])
    start_scat(NB - 2, ssems[(NB - 2) % 2])
    wait_g(NB - 1, gsems[(NB - 1) % 2])
    start_scat(NB - 1, ssems[(NB - 1) % 2])
    drain_scat(NB - 2, ssems[(NB - 2) % 2])
    drain_scat(NB - 1, ssems[(NB - 1) % 2])
    wait_idx(NB - 1, isems[(NB - 2) % 2])
    plsc.subcore_barrier()
    pltpu.sync_copy(acc_sh.at[pl.ds(s * RPS, RPS)],
                    out.at[c, pl.ds(s * RPS, RPS)])


_RB = 1000


def _scale_body(x_ref, dga_ref, dgb_ref, g_ref):
    deg = dga_ref[...] + dgb_ref[...] + 1.0
    g_ref[...] = x_ref[...] * lax.rsqrt(deg)


def _tc_scale(x, dga, dgb):
    return pl.pallas_call(
        _scale_body,
        grid=(N // _RB,),
        in_specs=[
            pl.BlockSpec((_RB, D), lambda i: (i, 0)),
            pl.BlockSpec((_RB, 1), lambda i: (i, 0)),
            pl.BlockSpec((_RB, 1), lambda i: (i, 0)),
        ],
        out_specs=pl.BlockSpec((_RB, D), lambda i: (i, 0)),
        out_shape=jax.ShapeDtypeStruct((N, D), jnp.float32),
    )(x, dga, dgb)


def _final_body(acca_ref, accb_ref, g_ref, dga_ref, dgb_ref, w_ref, b_ref,
                o_ref):
    deg = dga_ref[...] + dgb_ref[...] + 1.0
    t = (acca_ref[0] + accb_ref[0] + g_ref[...]) * lax.rsqrt(deg)
    h = jnp.dot(t, w_ref[...], preferred_element_type=jnp.float32)
    h = h + b_ref[...]
    val = h[:, :D]
    gate = h[:, D:]
    o_ref[...] = val * (0.5 * gate * (1.0 + lax.erf(gate * 0.7071067811865476)))


def _tc_final(accp, g, dga, dgb, W, b2):
    return pl.pallas_call(
        _final_body,
        grid=(N // _RB,),
        in_specs=[
            pl.BlockSpec((1, _RB, D), lambda i: (0, i, 0)),
            pl.BlockSpec((1, _RB, D), lambda i: (1, i, 0)),
            pl.BlockSpec((_RB, D), lambda i: (i, 0)),
            pl.BlockSpec((_RB, 1), lambda i: (i, 0)),
            pl.BlockSpec((_RB, 1), lambda i: (i, 0)),
            pl.BlockSpec((D, 2 * D), lambda i: (0, 0)),
            pl.BlockSpec((1, 2 * D), lambda i: (0, 0)),
        ],
        out_specs=pl.BlockSpec((_RB, D), lambda i: (i, 0)),
        out_shape=jax.ShapeDtypeStruct((N, D), jnp.float32),
    )(accp, accp, g, dga, dgb, W, b2)


def kernel(x, edge_index, W, b):
    npad = EP - E
    pad_src = jnp.zeros((npad,), jnp.int32)
    pad_dst = N + (jnp.arange(npad, dtype=jnp.int32) % (NP - N))
    src1d = jnp.concatenate([edge_index[0].astype(jnp.int32), pad_src])
    dst1d = jnp.concatenate([edge_index[1].astype(jnp.int32), pad_dst])
    ones1 = jnp.ones((B,), jnp.float32)
    zdeg = jnp.zeros((NP,), jnp.float32)
    zacc = jnp.zeros((NP, D), jnp.float32)

    degp = _deg_kernel(dst1d, ones1, zdeg)
    dga = degp[0].reshape(NP, 1)
    dgb = degp[1].reshape(NP, 1)
    g = _tc_scale(x, dga, dgb)
    accp = _agg_kernel(src1d, dst1d, g, zacc)
    return _tc_final(accp, g, dga, dgb, W, b.reshape(1, 2 * D))


# speedup vs baseline: 3.4382x; 3.4382x over previous
import functools

import jax
import jax.numpy as jnp
from jax import lax
from jax.experimental import pallas as pl
from jax.experimental.pallas import tpu as pltpu
from jax.experimental.pallas import tpu_sc as plsc

N = 10000
E = 320000
D = 128
DW = 16
NC, NS = 2, 16
NW = NC * NS
B = 80
NB = 125
EPW = NB * B
EP = NW * EPW
RPS = 640
NP = NS * RPS

_mesh = plsc.VectorSubcoreMesh(
    core_axis_name="c", subcore_axis_name="s", num_cores=NC, num_subcores=NS)


@functools.partial(
    pl.kernel,
    out_type=jax.ShapeDtypeStruct((NC, NP), jnp.float32),
    mesh=_mesh,
    scratch_types=[
        pltpu.VMEM_SHARED((NP,), jnp.float32),
        pltpu.VMEM((4, B), jnp.int32),
        pltpu.VMEM((B,), jnp.float32),
        pltpu.SemaphoreType.DMA,
        pltpu.SemaphoreType.DMA,
        pltpu.SemaphoreType.DMA,
        pltpu.SemaphoreType.DMA,
    ],
)
def _deg_kernel(dst1d, ones_hbm, zeros_hbm, out, deg_sh, didx4, ones_v,
                isem0, isem1, ssem0, ssem1):
    c = lax.axis_index("c")
    s = lax.axis_index("s")
    w = c * NS + s
    isems = (isem0, isem1)
    ssems = (ssem0, ssem1)
    pltpu.sync_copy(ones_hbm, ones_v)
    pltpu.sync_copy(zeros_hbm.at[pl.ds(s * RPS, RPS)],
                    deg_sh.at[pl.ds(s * RPS, RPS)])
    base = w * EPW

    def issue_idx(j, sem):
        pltpu.async_copy(
            dst1d.at[pl.ds(base + j * B, B)], didx4.at[lax.rem(j, 4)], sem)

    def wait_idx(j, sem):
        pltpu.make_async_copy(
            dst1d.at[pl.ds(base, B)], didx4.at[lax.rem(j, 4)], sem).wait()

    def start_scat(j, sem):
        pltpu.async_copy(
            ones_v, deg_sh.at[didx4.at[lax.rem(j, 4)]], sem, add=True)

    def drain_scat(j, sem):
        pltpu.make_async_copy(
            ones_v, deg_sh.at[didx4.at[lax.rem(j, 4)]], sem).wait()

    issue_idx(0, isem0)
    issue_idx(1, isem1)
    plsc.subcore_barrier()
    wait_idx(0, isem0)
    start_scat(0, ssem0)
    issue_idx(2, isem0)
    wait_idx(1, isem1)
    start_scat(1, ssem1)
    issue_idx(3, isem1)

    def body(j2, carry):
        for q in (0, 1):
            j = j2 * 2 + q
            jn = jnp.minimum(j + 2, NB - 1)
            drain_scat(j - 2, ssems[q])
            wait_idx(j, isems[q])
            start_scat(j, ssems[q])
            issue_idx(jn, isems[q])
        return carry

    lax.fori_loop(1, (NB - 1) // 2, body, None)
    drain_scat(NB - 3, ssems[(NB - 3) % 2])
    wait_idx(NB - 1, isems[(NB - 1) % 2])
    start_scat(NB - 1, ssems[(NB - 1) % 2])
    drain_scat(NB - 2, ssems[(NB - 2) % 2])
    drain_scat(NB - 1, ssems[(NB - 1) % 2])
    wait_idx(NB - 1, isems[(NB - 2) % 2])
    plsc.subcore_barrier()
    pltpu.sync_copy(deg_sh.at[pl.ds(s * RPS, RPS)],
                    out.at[c, pl.ds(s * RPS, RPS)])


@functools.partial(
    pl.kernel,
    out_type=jax.ShapeDtypeStruct((NC, NP, D), jnp.float32),
    mesh=_mesh,
    scratch_types=[
        pltpu.VMEM_SHARED((NP, D), jnp.float32),
        pltpu.VMEM((4, B), jnp.int32),
        pltpu.VMEM((4, B), jnp.int32),
        pltpu.VMEM((3, B, D), jnp.float32),
        pltpu.SemaphoreType.DMA,
        pltpu.SemaphoreType.DMA,
        pltpu.SemaphoreType.DMA,
        pltpu.SemaphoreType.DMA,
        pltpu.SemaphoreType.DMA,
        pltpu.SemaphoreType.DMA,
    ],
)
def _agg_kernel(src1d, dst1d, g_hbm, zeros_hbm, out, acc_sh, sidx4, didx4,
                rows3, isem0, isem1, gsem0, gsem1, ssem0, ssem1):
    c = lax.axis_index("c")
    s = lax.axis_index("s")
    w = c * NS + s
    isems = (isem0, isem1)
    gsems = (gsem0, gsem1)
    ssems = (ssem0, ssem1)
    base = w * EPW

    def issue_idx(j, sem):
        t = lax.rem(j, 4)
        pltpu.async_copy(src1d.at[pl.ds(base + j * B, B)], sidx4.at[t], sem)
        pltpu.async_copy(dst1d.at[pl.ds(base + j * B, B)], didx4.at[t], sem)

    def wait_idx(j, sem):
        t = lax.rem(j, 4)
        pltpu.make_async_copy(src1d.at[pl.ds(base, B)], sidx4.at[t],
                              sem).wait()
        pltpu.make_async_copy(dst1d.at[pl.ds(base, B)], didx4.at[t],
                              sem).wait()

    def issue_g(j, sem):
        pltpu.async_copy(
            g_hbm.at[sidx4.at[lax.rem(j, 4)]], rows3.at[lax.rem(j, 3)], sem)

    def wait_g(j, sem):
        pltpu.make_async_copy(
            g_hbm.at[sidx4.at[lax.rem(j, 4)]], rows3.at[lax.rem(j, 3)],
            sem).wait()

    def start_scat(j, sem):
        pltpu.async_copy(
            rows3.at[lax.rem(j, 3)], acc_sh.at[didx4.at[lax.rem(j, 4)]],
            sem, add=True)

    def drain_scat(j, sem):
        pltpu.make_async_copy(
            rows3.at[lax.rem(j, 3)], acc_sh.at[didx4.at[lax.rem(j, 4)]],
            sem).wait()

    issue_idx(0, isem0)
    issue_idx(1, isem1)
    pltpu.sync_copy(zeros_hbm.at[pl.ds(s * RPS, RPS)],
                    acc_sh.at[pl.ds(s * RPS, RPS)])
    plsc.subcore_barrier()

    wait_idx(0, isem0)
    issue_g(0, gsem0)
    issue_idx(2, isem0)
    wait_idx(1, isem1)
    issue_g(1, gsem1)
    issue_idx(3, isem1)
    wait_g(0, gsem0)
    start_scat(0, ssem0)
    wait_idx(2, isem0)
    issue_g(2, gsem0)
    wait_g(1, gsem1)

    def body(j2, carry):
        for q in (0, 1):
            j = j2 * 2 + q
            drain_scat(j - 2, ssems[q])
            start_scat(j - 1, ssems[1 - q])
            wait_idx(j + 1, isems[1 - q])
            issue_g(j + 1, gsems[1 - q])
            issue_idx(jnp.minimum(j + 2, NB - 1), isems[q])
            wait_g(j, gsems[q])
        return carry

    lax.fori_loop(1, (NB - 1) // 2, body, None)
    drain_scat(NB - 3, ssems[(NB - 3) % 2])
    start_scat(NB - 2, ssems[(NB - 2) % 2])
    wait_g(NB - 1, gsems[(NB - 1) % 2])
    start_scat(NB - 1, ssems[(NB - 1) % 2])
    drain_scat(NB - 2, ssems[(NB - 2) % 2])
    drain_scat(NB - 1, ssems[(NB - 1) % 2])
    wait_idx(NB - 1, isems[(NB - 2) % 2])
    plsc.subcore_barrier()
    pltpu.sync_copy(acc_sh.at[pl.ds(s * RPS, RPS)],
                    out.at[c, pl.ds(s * RPS, RPS)])


_RB = 1000


def _scale_body(x_ref, dga_ref, dgb_ref, g_ref):
    deg = dga_ref[...] + dgb_ref[...] + 1.0
    g_ref[...] = x_ref[...] * lax.rsqrt(deg)


def _tc_scale(x, dga, dgb):
    return pl.pallas_call(
        _scale_body,
        grid=(N // _RB,),
        in_specs=[
            pl.BlockSpec((_RB, D), lambda i: (i, 0)),
            pl.BlockSpec((_RB, 1), lambda i: (i, 0)),
            pl.BlockSpec((_RB, 1), lambda i: (i, 0)),
        ],
        out_specs=pl.BlockSpec((_RB, D), lambda i: (i, 0)),
        out_shape=jax.ShapeDtypeStruct((N, D), jnp.float32),
    )(x, dga, dgb)


def _final_body(acca_ref, accb_ref, g_ref, dga_ref, dgb_ref, w_ref, b_ref,
                o_ref):
    deg = dga_ref[...] + dgb_ref[...] + 1.0
    t = (acca_ref[0] + accb_ref[0] + g_ref[...]) * lax.rsqrt(deg)
    h = jnp.dot(t, w_ref[...], preferred_element_type=jnp.float32)
    h = h + b_ref[...]
    val = h[:, :D]
    gate = h[:, D:]
    o_ref[...] = val * (0.5 * gate * (1.0 + lax.erf(gate * 0.7071067811865476)))


def _tc_final(accp, g, dga, dgb, W, b2):
    return pl.pallas_call(
        _final_body,
        grid=(N // _RB,),
        in_specs=[
            pl.BlockSpec((1, _RB, D), lambda i: (0, i, 0)),
            pl.BlockSpec((1, _RB, D), lambda i: (1, i, 0)),
            pl.BlockSpec((_RB, D), lambda i: (i, 0)),
            pl.BlockSpec((_RB, 1), lambda i: (i, 0)),
            pl.BlockSpec((_RB, 1), lambda i: (i, 0)),
            pl.BlockSpec((D, 2 * D), lambda i: (0, 0)),
            pl.BlockSpec((1, 2 * D), lambda i: (0, 0)),
        ],
        out_specs=pl.BlockSpec((_RB, D), lambda i: (i, 0)),
        out_shape=jax.ShapeDtypeStruct((N, D), jnp.float32),
    )(accp, accp, g, dga, dgb, W, b2)


def kernel(x, edge_index, W, b):
    npad = EP - E
    pad_src = jnp.zeros((npad,), jnp.int32)
    pad_dst = N + (jnp.arange(npad, dtype=jnp.int32) % (NP - N))
    src1d = jnp.concatenate([edge_index[0].astype(jnp.int32), pad_src])
    dst1d = jnp.concatenate([edge_index[1].astype(jnp.int32), pad_dst])
    ones1 = jnp.ones((B,), jnp.float32)
    zdeg = jnp.zeros((NP,), jnp.float32)
    zacc = jnp.zeros((NP, D), jnp.float32)

    degp = _deg_kernel(dst1d, ones1, zdeg)
    dga = degp[0].reshape(NP, 1)
    dgb = degp[1].reshape(NP, 1)
    g = _tc_scale(x, dga, dgb)
    accp = _agg_kernel(src1d, dst1d, g, zacc)
    return _tc_final(accp, g, dga, dgb, W, b.reshape(1, 2 * D))
